# baseline (speedup 1.0000x reference)
import functools
import math

import jax
import jax.numpy as jnp
from jax.experimental import pallas as pl

N = 10000
E = 80000
H = 8
D = 32
HD = H * D
NODE_TYPES = ("text", "emotion", "intensity")
EDGE_TYPES = (("text", "emotion"), ("emotion", "text"), ("text", "intensity"), ("intensity", "text"))

_ROW_BLOCK = 2000


def _mm_kernel(x_ref, w_ref, b_ref, o_ref, *, act):
    acc = jnp.dot(x_ref[...], w_ref[...], preferred_element_type=jnp.float32)
    acc = acc + b_ref[...]
    if act == "relu":
        acc = jnp.maximum(acc, 0.0)
    o_ref[...] = acc


def _mm(x, w, b, act=None):
    n, k = x.shape
    kout = w.shape[1]
    grid = (n // _ROW_BLOCK,)
    return pl.pallas_call(
        functools.partial(_mm_kernel, act=act),
        grid=grid,
        in_specs=[
            pl.BlockSpec((_ROW_BLOCK, k), lambda i: (i, 0)),
            pl.BlockSpec((k, kout), lambda i: (0, 0)),
            pl.BlockSpec((1, kout), lambda i: (0, 0)),
        ],
        out_specs=pl.BlockSpec((_ROW_BLOCK, kout), lambda i: (i, 0)),
        out_shape=jax.ShapeDtypeStruct((n, kout), jnp.float32),
    )(x, w, b.reshape(1, kout))


def _out_kernel(m_ref, x_ref, w_ref, b_ref, beta_ref, o_ref):
    g = jax.nn.gelu(m_ref[...])
    acc = jnp.dot(g, w_ref[...], preferred_element_type=jnp.float32) + b_ref[...]
    beta = jax.nn.sigmoid(beta_ref[0, 0])
    o_ref[...] = beta * acc + (1.0 - beta) * x_ref[...]


def _out_lin(msg, x, w, b, beta):
    n, k = msg.shape
    grid = (n // _ROW_BLOCK,)
    return pl.pallas_call(
        _out_kernel,
        grid=grid,
        in_specs=[
            pl.BlockSpec((_ROW_BLOCK, k), lambda i: (i, 0)),
            pl.BlockSpec((_ROW_BLOCK, k), lambda i: (i, 0)),
            pl.BlockSpec((k, k), lambda i: (0, 0)),
            pl.BlockSpec((1, k), lambda i: (0, 0)),
            pl.BlockSpec((1, 1), lambda i: (0, 0)),
        ],
        out_specs=pl.BlockSpec((_ROW_BLOCK, k), lambda i: (i, 0)),
        out_shape=jax.ShapeDtypeStruct((n, k), jnp.float32),
    )(msg, x, w, b.reshape(1, k), beta.reshape(1, 1))


def _fold_rel(w, b, rel):
    w3 = w.reshape(w.shape[0], H, D)
    wf = jnp.einsum("khd,hde->khe", w3, rel).reshape(w.shape[0], HD)
    bf = jnp.einsum("hd,hde->he", b.reshape(H, D), rel).reshape(HD)
    return wf, bf


def _edge_phase(q_d, k_t, v_t, src, dst, p_rel):
    qd = q_d.reshape(N, H, D)
    kt = k_t.reshape(N, H, D)
    vt = v_t.reshape(N, H, D)
    alpha = (qd[dst] * kt[src]).sum(-1) * p_rel / math.sqrt(D)
    amax = jax.ops.segment_max(alpha, dst, num_segments=N)
    amax = jnp.where(jnp.isfinite(amax), amax, 0.0)
    e = jnp.exp(alpha - amax[dst])
    s = jax.ops.segment_sum(e, dst, num_segments=N)
    a_n = e / (s[dst] + 1e-16)
    msg = vt[src] * a_n[..., None]
    return jax.ops.segment_sum(msg, dst, num_segments=N).reshape(N, HD)


def kernel(x_text, x_emotion, x_intensity, edge_index_text__to__emotion,
           edge_index_emotion__to__text, edge_index_text__to__intensity,
           edge_index_intensity__to__text, params):
    edges = {
        "text__to__emotion": edge_index_text__to__emotion,
        "emotion__to__text": edge_index_emotion__to__text,
        "text__to__intensity": edge_index_text__to__intensity,
        "intensity__to__text": edge_index_intensity__to__text,
    }
    xr = {"text": x_text, "emotion": x_emotion, "intensity": x_intensity}
    x = {nt: _mm(xr[nt], params["lin_in"][nt]["W"], params["lin_in"][nt]["b"], act="relu")
         for nt in NODE_TYPES}

    for lp in params["layers"]:
        q = {nt: _mm(x[nt], lp["q"][nt]["W"], lp["q"][nt]["b"]) for nt in NODE_TYPES}
        agg = {nt: jnp.zeros((N, HD), jnp.float32) for nt in NODE_TYPES}
        for s_t, d_t in EDGE_TYPES:
            ek = s_t + "__to__" + d_t
            wk, bk = _fold_rel(lp["k"][s_t]["W"], lp["k"][s_t]["b"], lp["a_rel"][ek])
            wv, bv = _fold_rel(lp["v"][s_t]["W"], lp["v"][s_t]["b"], lp["m_rel"][ek])
            k_t = _mm(x[s_t], wk, bk)
            v_t = _mm(x[s_t], wv, bv)
            ei = edges[ek]
            agg[d_t] = agg[d_t] + _edge_phase(q[d_t], k_t, v_t, ei[0], ei[1], lp["p_rel"][ek])
        x = {nt: _out_lin(agg[nt], x[nt], lp["a"][nt]["W"], lp["a"][nt]["b"], lp["skip"][nt])
             for nt in NODE_TYPES}
    return (x["text"], x["emotion"], x["intensity"])


# speedup vs baseline: 1.0175x; 1.0175x over previous
import functools
import math

import jax
import jax.numpy as jnp
from jax import lax
from jax.experimental import pallas as pl

N = 10000
E = 80000
H = 8
D = 32
HD = H * D
NODE_TYPES = ("text", "emotion", "intensity")
EDGE_TYPES = (("text", "emotion"), ("emotion", "text"), ("text", "intensity"), ("intensity", "text"))

_ROW_BLOCK = 2000


def _mm_kernel(x_ref, w_ref, b_ref, o_ref, *, act):
    acc = jnp.dot(x_ref[...], w_ref[...], preferred_element_type=jnp.float32)
    acc = acc + b_ref[...]
    if act == "relu":
        acc = jnp.maximum(acc, 0.0)
    o_ref[...] = acc


def _mm(x, w, b, act=None):
    n, k = x.shape
    kout = w.shape[1]
    grid = (n // _ROW_BLOCK,)
    return pl.pallas_call(
        functools.partial(_mm_kernel, act=act),
        grid=grid,
        in_specs=[
            pl.BlockSpec((_ROW_BLOCK, k), lambda i: (i, 0)),
            pl.BlockSpec((k, kout), lambda i: (0, 0)),
            pl.BlockSpec((1, kout), lambda i: (0, 0)),
        ],
        out_specs=pl.BlockSpec((_ROW_BLOCK, kout), lambda i: (i, 0)),
        out_shape=jax.ShapeDtypeStruct((n, kout), jnp.float32),
    )(x, w, b.reshape(1, kout))


def _out_kernel(*refs, n_msg):
    msg_refs, (x_ref, w_ref, b_ref, beta_ref, o_ref) = refs[:4 * n_msg], refs[4 * n_msg:]
    nrows = x_ref.shape[0]
    exp_mat = (lax.broadcasted_iota(jnp.int32, (16, HD), 1) // D
               == lax.broadcasted_iota(jnp.int32, (16, HD), 0)).astype(jnp.float32)
    m = jnp.zeros((nrows, HD), jnp.float32)
    for j in range(n_msg):
        u0, u1, s0, s1 = (msg_refs[4 * j][...], msg_refs[4 * j + 1][...],
                          msg_refs[4 * j + 2][...], msg_refs[4 * j + 3][...])
        s16 = s0 + s1
        denom = jnp.dot(s16, exp_mat, preferred_element_type=jnp.float32) + 1e-16
        m = m + jnp.concatenate([u0, u1], axis=-1) / denom
    g = jax.nn.gelu(m)
    acc = jnp.dot(g, w_ref[...], preferred_element_type=jnp.float32) + b_ref[...]
    beta = jax.nn.sigmoid(beta_ref[0, 0])
    o_ref[...] = beta * acc + (1.0 - beta) * x_ref[...]


def _out_lin(msgs, x, w, b, beta):
    n, k = x.shape
    nb = n // _ROW_BLOCK
    grid = (nb,)
    blk = pl.BlockSpec((_ROW_BLOCK, k), lambda i: (i, 0))
    flat, specs = [], []
    for u, sarr in msgs:
        flat += [u, u, sarr, sarr]
        specs += [
            pl.BlockSpec((_ROW_BLOCK, k // 2), lambda i: (i, 0)),
            pl.BlockSpec((_ROW_BLOCK, k // 2), lambda i, _nb=nb: (i + _nb, 0)),
            pl.BlockSpec((_ROW_BLOCK, 16), lambda i: (i, 0)),
            pl.BlockSpec((_ROW_BLOCK, 16), lambda i, _nb=nb: (i + _nb, 0)),
        ]
    return pl.pallas_call(
        functools.partial(_out_kernel, n_msg=len(msgs)),
        grid=grid,
        in_specs=specs + [
            blk,
            pl.BlockSpec((k, k), lambda i: (0, 0)),
            pl.BlockSpec((1, k), lambda i: (0, 0)),
            pl.BlockSpec((1, 1), lambda i: (0, 0)),
        ],
        out_specs=blk,
        out_shape=jax.ShapeDtypeStruct((n, k), jnp.float32),
    )(*flat, x, w, b.reshape(1, k), beta.reshape(1, 1))


def _fold_rel(w, b, rel, head_scale=None):
    if head_scale is not None:
        rel = rel * head_scale[:, None, None]
    w3 = w.reshape(w.shape[0], H, D)
    wf = jnp.einsum("khd,hde->khe", w3, rel).reshape(w.shape[0], HD)
    bf = jnp.einsum("hd,hde->he", b.reshape(H, D), rel).reshape(HD)
    return wf, bf


def _edge_phase(q_d, k_t, v_t, src, dst):
    qd = q_d.reshape(N, H, D)
    kt = k_t.reshape(N, H, D)
    vt = v_t.reshape(N, H, D)
    alpha = (qd[dst] * kt[src]).sum(-1)
    amax = jax.ops.segment_max(alpha, dst, num_segments=N)
    amax = jnp.where(jnp.isfinite(amax), amax, 0.0)
    e = jnp.exp(alpha - amax[dst])
    ssum = jax.ops.segment_sum(e, dst, num_segments=N)
    u = jax.ops.segment_sum(vt[src] * e[..., None], dst, num_segments=N)
    u = u.reshape(N, HD)
    u2 = jnp.concatenate([u[:, :HD // 2], u[:, HD // 2:]], axis=0)
    s0 = jnp.pad(ssum[:, :4], ((0, 0), (0, 12)))
    s1 = jnp.pad(ssum[:, 4:], ((0, 0), (4, 8)))
    s2 = jnp.concatenate([s0, s1], axis=0)
    return u2, s2


def kernel(x_text, x_emotion, x_intensity, edge_index_text__to__emotion,
           edge_index_emotion__to__text, edge_index_text__to__intensity,
           edge_index_intensity__to__text, params):
    edges = {
        "text__to__emotion": edge_index_text__to__emotion,
        "emotion__to__text": edge_index_emotion__to__text,
        "text__to__intensity": edge_index_text__to__intensity,
        "intensity__to__text": edge_index_intensity__to__text,
    }
    xr = {"text": x_text, "emotion": x_emotion, "intensity": x_intensity}
    x = {nt: _mm(xr[nt], params["lin_in"][nt]["W"], params["lin_in"][nt]["b"], act="relu")
         for nt in NODE_TYPES}

    for lp in params["layers"]:
        q = {nt: _mm(x[nt], lp["q"][nt]["W"], lp["q"][nt]["b"]) for nt in NODE_TYPES}
        aggs = {nt: [] for nt in NODE_TYPES}
        for s_t, d_t in EDGE_TYPES:
            ek = s_t + "__to__" + d_t
            wk, bk = _fold_rel(lp["k"][s_t]["W"], lp["k"][s_t]["b"], lp["a_rel"][ek],
                               head_scale=lp["p_rel"][ek] / math.sqrt(D))
            wv, bv = _fold_rel(lp["v"][s_t]["W"], lp["v"][s_t]["b"], lp["m_rel"][ek])
            k_t = _mm(x[s_t], wk, bk)
            v_t = _mm(x[s_t], wv, bv)
            ei = edges[ek]
            aggs[d_t].append(_edge_phase(q[d_t], k_t, v_t, ei[0], ei[1]))
        x = {nt: _out_lin(aggs[nt], x[nt], lp["a"][nt]["W"], lp["a"][nt]["b"], lp["skip"][nt])
             for nt in NODE_TYPES}
    return (x["text"], x["emotion"], x["intensity"])
